# Initial kernel scaffold; baseline (speedup 1.0000x reference)
import jax
import jax.numpy as jnp
from jax.experimental import pallas as pl


def kernel(phase, amplitude, frequencies, mu, neighbors):
    raise NotImplementedError("write your pallas kernel here")


# speedup vs baseline: 4.9454x; 4.9454x over previous
import functools
import math

import jax
import jax.numpy as jnp
from jax import lax
from jax.experimental import pallas as pl
from jax.experimental.pallas import tpu as pltpu
from jax.experimental.pallas import tpu_sc as plsc

DT = 0.01
COUPLING_STRENGTH = 2.0
TWO_PI = 2.0 * math.pi

NUM_CORES = 2
NUM_SUBCORES = 16
LANES = 16

B, N, K = 64, 10000, 16
BPW = B // NUM_SUBCORES
HALF = N // NUM_CORES
W = 1024
SLAB_STARTS = (0, 1024, 2048, 3072, HALF - W)


def _tc_elemwise_body(mu_ref, phase_ref, amp_ref, s_ref, c_ref, amp_out_ref):
    p = phase_ref[...]
    s_ref[...] = jnp.sin(p)
    c_ref[...] = jnp.cos(p)
    a = amp_ref[...]
    mu = mu_ref[0]
    na = a + DT * a * (mu - a * a)
    amp_out_ref[...] = jnp.clip(na, 1e-06, 10.0)


def _tc_elemwise(phase, amplitude, mu):
    out_shape = [
        jax.ShapeDtypeStruct((B, N), jnp.float32),
        jax.ShapeDtypeStruct((B, N), jnp.float32),
        jax.ShapeDtypeStruct((B, N), jnp.float32),
    ]
    return pl.pallas_call(
        _tc_elemwise_body,
        out_shape=out_shape,
        in_specs=[
            pl.BlockSpec(memory_space=pltpu.SMEM),
            pl.BlockSpec(memory_space=pltpu.VMEM),
            pl.BlockSpec(memory_space=pltpu.VMEM),
        ],
        out_specs=[pl.BlockSpec(memory_space=pltpu.VMEM)] * 3,
    )(jnp.reshape(mu.astype(jnp.float32), (1,)), phase, amplitude)


def _sc_body(s_hbm, c_hbm, phase_hbm, freq_hbm, nbrt_hbm, out_hbm,
             s_tab, c_tab, idx_b, ph_b, fr_b, out_b, sem):
    cid = lax.axis_index("c")
    sid = lax.axis_index("s")
    b0 = sid * BPW

    tab_cps = []
    for i in range(BPW):
        tab_cps.append(pltpu.async_copy(s_hbm.at[b0 + i], s_tab.at[i], sem))
        tab_cps.append(pltpu.async_copy(c_hbm.at[b0 + i], c_tab.at[i], sem))
    for cp in tab_cps:
        cp.wait()

    for slab_start in SLAB_STARTS:
        start = cid * HALF + slab_start

        cps = []
        for k in range(K):
            cps.append(pltpu.async_copy(
                nbrt_hbm.at[k, pl.ds(start, W)], idx_b.at[k], sem))
        for i in range(BPW):
            cps.append(pltpu.async_copy(
                phase_hbm.at[b0 + i, pl.ds(start, W)], ph_b.at[i], sem))
        cps.append(pltpu.async_copy(freq_hbm.at[pl.ds(start, W)], fr_b, sem))
        for cp in cps:
            cp.wait()

        def block_body(blk, carry):
            nl = blk * LANES
            gcol = start + nl
            ivs = [idx_b[k, pl.ds(nl, LANES)] for k in range(K)]
            for b in range(BPW):
                bv = jnp.full((LANES,), b, jnp.int32)
                acc_s = plsc.load_gather(s_tab, [bv, ivs[0]])
                acc_c = plsc.load_gather(c_tab, [bv, ivs[0]])
                for k in range(1, K):
                    acc_s = acc_s + plsc.load_gather(s_tab, [bv, ivs[k]])
                    acc_c = acc_c + plsc.load_gather(c_tab, [bv, ivs[k]])
                sv = s_tab[b, pl.ds(gcol, LANES)]
                cv = c_tab[b, pl.ds(gcol, LANES)]
                coup = (COUPLING_STRENGTH / K) * (cv * acc_s - sv * acc_c)
                pv = ph_b[b, pl.ds(nl, LANES)]
                fv = fr_b[pl.ds(nl, LANES)]
                t = pv + (TWO_PI * fv) * DT + DT * coup
                r = lax.rem(t, TWO_PI)
                r = jnp.where(r < 0.0, r + TWO_PI, r)
                out_b[b, pl.ds(nl, LANES)] = r
            return carry

        lax.fori_loop(0, W // LANES, block_body, 0)

        out_cps = []
        for i in range(BPW):
            out_cps.append(pltpu.async_copy(
                out_b.at[i], out_hbm.at[b0 + i, pl.ds(start, W)], sem))
        for cp in out_cps:
            cp.wait()


def _sc_gather(s, c, phase, freq, nbrt):
    mesh = plsc.VectorSubcoreMesh(
        core_axis_name="c", subcore_axis_name="s",
        num_cores=NUM_CORES, num_subcores=NUM_SUBCORES)
    return pl.kernel(
        _sc_body,
        out_type=jax.ShapeDtypeStruct((B, N), jnp.float32),
        mesh=mesh,
        compiler_params=pltpu.CompilerParams(
            use_tc_tiling_on_sc=False, needs_layout_passes=False),
        scratch_types=[
            pltpu.VMEM((BPW, N), jnp.float32),
            pltpu.VMEM((BPW, N), jnp.float32),
            pltpu.VMEM((K, W), jnp.int32),
            pltpu.VMEM((BPW, W), jnp.float32),
            pltpu.VMEM((W,), jnp.float32),
            pltpu.VMEM((BPW, W), jnp.float32),
            pltpu.SemaphoreType.DMA,
        ],
    )(s, c, phase, freq, nbrt)


def kernel(phase, amplitude, frequencies, mu, neighbors):
    nbrt = neighbors.T
    s, c, new_amp = _tc_elemwise(phase, amplitude, mu)
    new_phase = _sc_gather(s, c, phase, frequencies, nbrt)
    return new_phase, new_amp


# speedup vs baseline: 5.6186x; 1.1361x over previous
import functools
import math

import jax
import jax.numpy as jnp
from jax import lax
from jax.experimental import pallas as pl
from jax.experimental.pallas import tpu as pltpu
from jax.experimental.pallas import tpu_sc as plsc

DT = 0.01
COUPLING_STRENGTH = 2.0
TWO_PI = 2.0 * math.pi
INV_TWO_PI = 1.0 / TWO_PI

NUM_CORES = 2
NUM_SUBCORES = 16
LANES = 16

B, N, K = 64, 10000, 16
BPW = B // NUM_SUBCORES
HALF = N // NUM_CORES
W = 1024
SLAB_STARTS = (0, 1024, 2048, 3072, HALF - W)


def _tc_elemwise_body(mu_ref, phase_ref, amp_ref, freq_ref,
                      s_ref, c_ref, base_ref, amp_out_ref):
    p = phase_ref[...]
    s_ref[...] = jnp.sin(p)
    c_ref[...] = jnp.cos(p)
    f = freq_ref[...]
    base_ref[...] = p + (TWO_PI * f) * DT
    a = amp_ref[...]
    mu = mu_ref[0]
    na = a + DT * a * (mu - a * a)
    amp_out_ref[...] = jnp.clip(na, 1e-06, 10.0)


def _tc_elemwise(phase, amplitude, frequencies, mu):
    out_shape = [
        jax.ShapeDtypeStruct((B, N), jnp.float32),
        jax.ShapeDtypeStruct((B, N), jnp.float32),
        jax.ShapeDtypeStruct((B, N), jnp.float32),
        jax.ShapeDtypeStruct((B, N), jnp.float32),
    ]
    return pl.pallas_call(
        _tc_elemwise_body,
        out_shape=out_shape,
        in_specs=[
            pl.BlockSpec(memory_space=pltpu.SMEM),
            pl.BlockSpec(memory_space=pltpu.VMEM),
            pl.BlockSpec(memory_space=pltpu.VMEM),
            pl.BlockSpec(memory_space=pltpu.VMEM),
        ],
        out_specs=[pl.BlockSpec(memory_space=pltpu.VMEM)] * 4,
    )(jnp.reshape(mu.astype(jnp.float32), (1,)), phase, amplitude,
      jnp.reshape(frequencies, (1, N)))


def _sc_body(s_hbm, c_hbm, base_hbm, nbrt_hbm, out_hbm,
             s_tab, c_tab, idx_b, base_b, out_b, sem):
    cid = lax.axis_index("c")
    sid = lax.axis_index("s")
    b0 = sid * BPW

    tab_cps = []
    for i in range(BPW):
        tab_cps.append(pltpu.async_copy(s_hbm.at[b0 + i], s_tab.at[i], sem))
        tab_cps.append(pltpu.async_copy(c_hbm.at[b0 + i], c_tab.at[i], sem))
    for cp in tab_cps:
        cp.wait()

    for slab_start in SLAB_STARTS:
        start = cid * HALF + slab_start

        cps = []
        for k in range(K):
            cps.append(pltpu.async_copy(
                nbrt_hbm.at[k, pl.ds(start, W)], idx_b.at[k], sem))
        for i in range(BPW):
            cps.append(pltpu.async_copy(
                base_hbm.at[b0 + i, pl.ds(start, W)], base_b.at[i], sem))
        for cp in cps:
            cp.wait()

        @plsc.parallel_loop(0, W // LANES)
        def block_body(blk):
            nl = blk * LANES
            gcol = start + nl
            ivs = [idx_b[k, pl.ds(nl, LANES)] for k in range(K)]
            for b in range(BPW):
                bv = jnp.full((LANES,), b, jnp.int32)
                acc_s = plsc.load_gather(s_tab, [bv, ivs[0]])
                acc_c = plsc.load_gather(c_tab, [bv, ivs[0]])
                for k in range(1, K):
                    acc_s = acc_s + plsc.load_gather(s_tab, [bv, ivs[k]])
                    acc_c = acc_c + plsc.load_gather(c_tab, [bv, ivs[k]])
                sv = s_tab[b, pl.ds(gcol, LANES)]
                cv = c_tab[b, pl.ds(gcol, LANES)]
                coup = (COUPLING_STRENGTH / K) * (cv * acc_s - sv * acc_c)
                t = base_b[b, pl.ds(nl, LANES)] + DT * coup
                q0 = t * INV_TWO_PI
                qf = q0.astype(jnp.int32).astype(jnp.float32)
                q = jnp.where(qf > q0, qf - 1.0, qf)
                out_b[b, pl.ds(nl, LANES)] = t - q * TWO_PI

        out_cps = []
        for i in range(BPW):
            out_cps.append(pltpu.async_copy(
                out_b.at[i], out_hbm.at[b0 + i, pl.ds(start, W)], sem))
        for cp in out_cps:
            cp.wait()


def _sc_gather(s, c, base, nbrt):
    mesh = plsc.VectorSubcoreMesh(
        core_axis_name="c", subcore_axis_name="s",
        num_cores=NUM_CORES, num_subcores=NUM_SUBCORES)
    return pl.kernel(
        _sc_body,
        out_type=jax.ShapeDtypeStruct((B, N), jnp.float32),
        mesh=mesh,
        compiler_params=pltpu.CompilerParams(
            use_tc_tiling_on_sc=False, needs_layout_passes=False),
        scratch_types=[
            pltpu.VMEM((BPW, N), jnp.float32),
            pltpu.VMEM((BPW, N), jnp.float32),
            pltpu.VMEM((K, W), jnp.int32),
            pltpu.VMEM((BPW, W), jnp.float32),
            pltpu.VMEM((BPW, W), jnp.float32),
            pltpu.SemaphoreType.DMA,
        ],
    )(s, c, base, nbrt)


def kernel(phase, amplitude, frequencies, mu, neighbors):
    nbrt = neighbors.T
    s, c, base, new_amp = _tc_elemwise(phase, amplitude, frequencies, mu)
    new_phase = _sc_gather(s, c, base, nbrt)
    return new_phase, new_amp
